# Initial kernel scaffold; baseline (speedup 1.0000x reference)
import jax
import jax.numpy as jnp
from jax.experimental import pallas as pl


def kernel(x, pe):
    raise NotImplementedError("write your pallas kernel here")


# speedup vs baseline: 1.7221x; 1.7221x over previous
import jax
import jax.numpy as jnp
from jax.experimental import pallas as pl


def _add_kernel(x_ref, pe_ref, o_ref):
    o_ref[...] = x_ref[...] + pe_ref[...]


def kernel(x, pe):
    B, S, D = x.shape
    Sb = 512
    return pl.pallas_call(
        _add_kernel,
        grid=(S // Sb,),
        in_specs=[
            pl.BlockSpec((B, Sb, D), lambda i: (0, i, 0)),
            pl.BlockSpec((Sb, D), lambda i: (i, 0)),
        ],
        out_specs=pl.BlockSpec((B, Sb, D), lambda i: (0, i, 0)),
        out_shape=jax.ShapeDtypeStruct((B, S, D), x.dtype),
    )(x, pe[:S])
